# Initial kernel scaffold; baseline (speedup 1.0000x reference)
import jax
import jax.numpy as jnp
from jax.experimental import pallas as pl


def kernel(hiddens, seq_lens, intent_pro, slot_pro, intent_embedding, slot_embedding, gat_W, gat_a, gat_Wout, gat_aout, intent_W, intent_b, slot_W, slot_b):
    raise NotImplementedError("write your pallas kernel here")


# speedup vs baseline: 9.4805x; 9.4805x over previous
import functools

import jax
import jax.numpy as jnp
import numpy as np
from jax.experimental import pallas as pl
from jax.experimental.pallas import tpu as pltpu

B = 8; S = 512; H = 128; INTENT = 128; SLOT = 512
GHD = 16; GOD = 128; NHEAD = 4; TOPK = 3; WINDOW = 2; ALPHA = 0.2
N = S + INTENT + SLOT
NEG = -9e15
F32 = jnp.float32


def _lrelu(x):
    return jnp.where(x >= 0, x, ALPHA * x)


def _elu(x):
    return jnp.where(x > 0, x, jnp.exp(jnp.minimum(x, 0.0)) - 1.0)


def _bt_matmul(a, b):
    return jax.lax.dot_general(a, b, (((1,), (1,)), ((), ())),
                               preferred_element_type=F32)


def _mm(a, b):
    return jnp.dot(a, b, preferred_element_type=F32)


def _top3(x, width):
    cc = jax.lax.broadcasted_iota(jnp.int32, x.shape, 1)
    onehot = jnp.zeros(x.shape, F32)
    idx_cols = []
    for _ in range(TOPK):
        m = jnp.max(x, axis=1, keepdims=True)
        eq = x >= m
        idx = jnp.min(jnp.where(eq, cc, width), axis=1, keepdims=True)
        sel = cc == idx
        onehot = jnp.maximum(onehot, sel.astype(F32))
        idx_cols.append(idx)
        x = jnp.where(sel, -3.0e38, x)
    return idx_cols, onehot


def _masked_softmax_mm(mask, e, h):
    logits = jnp.where(mask, e, NEG)
    m = jnp.max(logits, axis=1, keepdims=True)
    p = jnp.exp(logits - m)
    s = jnp.sum(p, axis=1, keepdims=True)
    return _mm(p, h) / s


def _fused_body(hid_ref, ip_ref, sp_ref, iemb_ref, semb_ref, wcat_ref,
                a1m_ref, a2m_ref, wout_ref, aoutc_ref, iw_ref, ib_ref,
                sw_ref, sb_ref, hidden_ref, iout_ref, sout_ref):
    hid = hid_ref[0]
    ip = ip_ref[0]
    sp = sp_ref[0]

    ii_cols, si_oh_T = _top3(ip, INTENT)
    si_cols, ss_oh_T = _top3(sp, SLOT)
    int_oh_T = si_oh_T
    slot_oh_T = ss_oh_T

    rr = jax.lax.broadcasted_iota(jnp.int32, (S, N), 0)
    cc = jax.lax.broadcasted_iota(jnp.int32, (S, N), 1)
    band = (cc >= rr - WINDOW) & (cc <= rr + WINDOW)
    m_seq = band
    for k in range(TOPK):
        m_seq = m_seq | (cc == S + ii_cols[k])
        m_seq = m_seq | (cc == S + INTENT + si_cols[k])

    eyeI = (jax.lax.broadcasted_iota(jnp.int32, (INTENT, INTENT), 0) ==
            jax.lax.broadcasted_iota(jnp.int32, (INTENT, INTENT), 1))
    eyeS = (jax.lax.broadcasted_iota(jnp.int32, (SLOT, SLOT), 0) ==
            jax.lax.broadcasted_iota(jnp.int32, (SLOT, SLOT), 1))
    si_f = _bt_matmul(eyeI.astype(F32), int_oh_T)
    ss_f = _bt_matmul(eyeS.astype(F32), slot_oh_T)
    ri = _mm(si_f, slot_oh_T)
    rs = _mm(ss_f, int_oh_T)

    m_int = jnp.concatenate([si_f > 0.5, eyeI, ri > 0.5], axis=1)
    m_slot = jnp.concatenate([ss_f > 0.5, rs > 0.5, eyeS], axis=1)
    mask = jnp.concatenate([m_seq, m_int, m_slot], axis=0)

    hcat = jnp.concatenate([hid, iemb_ref[...], semb_ref[...]], axis=0)
    h_all = _mm(hcat, wcat_ref[...])
    a1_all = _mm(h_all, a1m_ref[...])
    a2_all = _mm(h_all, a2m_ref[...])
    ones_n = jnp.ones((N, 1), F32)

    head_outs = []
    for i in range(NHEAD):
        h_i = h_all[:, i * GHD:(i + 1) * GHD]
        lhs = jnp.concatenate([a1_all[:, i:i + 1], ones_n], axis=1)
        rhs = jnp.concatenate([ones_n, a2_all[:, i:i + 1]], axis=1)
        e = _lrelu(_bt_matmul(lhs, rhs))
        head_outs.append(_elu(_masked_softmax_mm(mask, e, h_i)))
    x2 = jnp.concatenate(head_outs, axis=1)

    h2 = _mm(x2, wout_ref[...])
    a1_2 = _mm(h2, aoutc_ref[...][:, 0:1])
    a2_2 = _mm(h2, aoutc_ref[...][:, 1:2])
    ones_s = jnp.ones((S, 1), F32)
    lhs2 = jnp.concatenate([a1_2[:S], ones_s], axis=1)
    rhs2 = jnp.concatenate([ones_n, a2_2], axis=1)
    e2 = _lrelu(_bt_matmul(lhs2, rhs2))
    hidden = _elu(_masked_softmax_mm(mask[:S], e2, h2))

    hidden_ref[0] = hidden
    iout_ref[0] = _mm(hidden, iw_ref[...]) + ib_ref[...]
    sout_ref[0] = _mm(hidden, sw_ref[...]) + sb_ref[...]


@jax.jit
def _run(hiddens, intent_pro, slot_pro, intent_embedding, slot_embedding,
         gat_W, gat_a, gat_Wout, gat_aout, intent_W, intent_b, slot_W, slot_b):
    wcat = jnp.transpose(gat_W, (1, 0, 2)).reshape(H, NHEAD * GHD)
    sel = np.arange(NHEAD).repeat(GHD)
    blkmask = jnp.asarray(sel[:, None] == np.arange(NHEAD)[None, :], F32)
    a1m = blkmask * gat_a[:, :GHD].reshape(-1, 1)
    a2m = blkmask * gat_a[:, GHD:].reshape(-1, 1)
    aoutc = jnp.stack([gat_aout[:GOD], gat_aout[GOD:]], axis=1)

    grid = (B,)
    fullmap = lambda b: (0, 0)
    out_shapes = (
        jax.ShapeDtypeStruct((B, S, GOD), F32),
        jax.ShapeDtypeStruct((B, S, INTENT), F32),
        jax.ShapeDtypeStruct((B, S, SLOT), F32),
    )
    outs = pl.pallas_call(
        _fused_body,
        grid=grid,
        in_specs=[
            pl.BlockSpec((1, S, H), lambda b: (b, 0, 0)),
            pl.BlockSpec((1, S, INTENT), lambda b: (b, 0, 0)),
            pl.BlockSpec((1, S, SLOT), lambda b: (b, 0, 0)),
            pl.BlockSpec((INTENT, H), fullmap),
            pl.BlockSpec((SLOT, H), fullmap),
            pl.BlockSpec((H, NHEAD * GHD), fullmap),
            pl.BlockSpec((NHEAD * GHD, NHEAD), fullmap),
            pl.BlockSpec((NHEAD * GHD, NHEAD), fullmap),
            pl.BlockSpec((NHEAD * GHD, GOD), fullmap),
            pl.BlockSpec((GOD, 2), fullmap),
            pl.BlockSpec((GOD, INTENT), fullmap),
            pl.BlockSpec((1, INTENT), fullmap),
            pl.BlockSpec((GOD, SLOT), fullmap),
            pl.BlockSpec((1, SLOT), fullmap),
        ],
        out_specs=(
            pl.BlockSpec((1, S, GOD), lambda b: (b, 0, 0)),
            pl.BlockSpec((1, S, INTENT), lambda b: (b, 0, 0)),
            pl.BlockSpec((1, S, SLOT), lambda b: (b, 0, 0)),
        ),
        out_shape=out_shapes,
    )(hiddens, intent_pro, slot_pro, intent_embedding, slot_embedding,
      wcat, a1m, a2m, gat_Wout, aoutc, intent_W,
      intent_b.reshape(1, INTENT), slot_W, slot_b.reshape(1, SLOT))
    return outs


def kernel(hiddens, seq_lens, intent_pro, slot_pro, intent_embedding,
           slot_embedding, gat_W, gat_a, gat_Wout, gat_aout, intent_W,
           intent_b, slot_W, slot_b):
    hidden, intent_out, slot_out = _run(
        hiddens, intent_pro, slot_pro, intent_embedding, slot_embedding,
        gat_W, gat_a, gat_Wout, gat_aout, intent_W, intent_b, slot_W, slot_b)
    return (hidden, hidden, intent_out, slot_out)
